# baseline (speedup 1.0000x reference)
import functools

import jax
import jax.numpy as jnp
from jax import lax
from jax.experimental import pallas as pl
from jax.experimental.pallas import tpu as pltpu
from jax.experimental.pallas import tpu_sc as plsc

E = 8
K = 2
H = 1024
F = 512
TB = 512
BM = 256
NB = 40
NBP = 48


def _router_body(x_ref, rw_ref, logits_ref, w_ref, e_ref, r_ref, start_ref,
                 be_ref, carry):
    i = pl.program_id(0)

    @pl.when(i == 0)
    def _():
        carry[...] = jnp.zeros_like(carry)

    x = x_ref[...]
    logits = jnp.dot(x, rw_ref[...], preferred_element_type=jnp.float32)
    logits_ref[...] = logits
    p = jax.nn.softmax(logits, axis=-1)
    iota8 = jax.lax.broadcasted_iota(jnp.int32, (TB, E), 1)
    m0 = jnp.max(p, axis=-1, keepdims=True)
    e0 = jnp.min(jnp.where(p == m0, iota8, E), axis=-1, keepdims=True)
    pm = jnp.where(iota8 == e0, -1.0, p)
    m1 = jnp.max(pm, axis=-1, keepdims=True)
    e1 = jnp.min(jnp.where(pm == m1, iota8, E), axis=-1, keepdims=True)
    s = m0 + m1
    w_ref[...] = jnp.concatenate([m0 / s, m1 / s], axis=-1)
    e_ref[...] = jnp.concatenate([e0, e1], axis=-1)

    oh0 = (iota8 == e0).astype(jnp.float32)
    oh1 = (iota8 == e1).astype(jnp.float32)
    tril = (jax.lax.broadcasted_iota(jnp.int32, (TB, TB), 0)
            > jax.lax.broadcasted_iota(jnp.int32, (TB, TB), 1)).astype(jnp.float32)
    c = carry[...]
    r0 = jnp.dot(tril, oh0, preferred_element_type=jnp.float32) + c
    c = c + jnp.sum(oh0, axis=0, keepdims=True)
    r1 = jnp.dot(tril, oh1, preferred_element_type=jnp.float32) + c
    c = c + jnp.sum(oh1, axis=0, keepdims=True)
    carry[...] = c
    rank0 = jnp.sum(oh0 * r0, axis=-1, keepdims=True)
    rank1 = jnp.sum(oh1 * r1, axis=-1, keepdims=True)
    r_ref[...] = jnp.concatenate([rank0, rank1], axis=-1).astype(jnp.int32)

    @pl.when(i == pl.num_programs(0) - 1)
    def _():
        padded = jnp.floor((c + (BM - 1)) / BM) * BM
        upper = (jax.lax.broadcasted_iota(jnp.int32, (E, E), 0)
                 <= jax.lax.broadcasted_iota(jnp.int32, (E, E), 1)
                 ).astype(jnp.float32)
        incl = jnp.dot(padded, upper, preferred_element_type=jnp.float32)
        starts = incl - padded
        start_ref[...] = jnp.concatenate(
            [starts, jnp.zeros((1, E), jnp.float32)], axis=-1).astype(jnp.int32)
        blk = (jax.lax.broadcasted_iota(jnp.int32, (NBP, E), 0)
               * BM).astype(jnp.float32)
        be_ref[...] = jnp.sum((blk >= incl).astype(jnp.int32),
                              axis=-1, keepdims=True)


def _router(x2d, router_w):
    t = x2d.shape[0]
    nblk = t // TB
    return pl.pallas_call(
        _router_body,
        grid=(nblk,),
        in_specs=[
            pl.BlockSpec((TB, H), lambda i: (i, 0)),
            pl.BlockSpec((H, E), lambda i: (0, 0)),
        ],
        out_specs=[
            pl.BlockSpec((TB, E), lambda i: (i, 0)),
            pl.BlockSpec((TB, K), lambda i: (i, 0)),
            pl.BlockSpec((TB, K), lambda i: (i, 0)),
            pl.BlockSpec((TB, K), lambda i: (i, 0)),
            pl.BlockSpec((1, 2 * E), lambda i: (0, 0)),
            pl.BlockSpec((NBP, 1), lambda i: (0, 0)),
        ],
        out_shape=[
            jax.ShapeDtypeStruct((t, E), jnp.float32),
            jax.ShapeDtypeStruct((t, K), jnp.float32),
            jax.ShapeDtypeStruct((t, K), jnp.int32),
            jax.ShapeDtypeStruct((t, K), jnp.int32),
            jax.ShapeDtypeStruct((1, 2 * E), jnp.int32),
            jax.ShapeDtypeStruct((NBP, 1), jnp.int32),
        ],
        scratch_shapes=[pltpu.VMEM((1, E), jnp.float32)],
    )(x2d, router_w)


def _mlp_body(be_ref, xs_ref, wg_ref, wu_ref, wd_ref, ys_ref):
    @pl.when(be_ref[pl.program_id(0)] < E)
    def _():
        x = xs_ref[...]
        g = jnp.dot(x, wg_ref[0], preferred_element_type=jnp.float32)
        u = jnp.dot(x, wu_ref[0], preferred_element_type=jnp.float32)
        h = (jax.nn.silu(g) * u).astype(jnp.bfloat16)
        ys_ref[...] = jnp.dot(h, wd_ref[0], preferred_element_type=jnp.float32)


def _grouped_mlp(block_expert, xs, Wg, Wu, Wd):
    grid_spec = pltpu.PrefetchScalarGridSpec(
        num_scalar_prefetch=1,
        grid=(NB,),
        in_specs=[
            pl.BlockSpec((BM, H), lambda i, be: (i, 0)),
            pl.BlockSpec((1, H, F), lambda i, be: (jnp.minimum(be[i], E - 1), 0, 0)),
            pl.BlockSpec((1, H, F), lambda i, be: (jnp.minimum(be[i], E - 1), 0, 0)),
            pl.BlockSpec((1, F, H), lambda i, be: (jnp.minimum(be[i], E - 1), 0, 0)),
        ],
        out_specs=pl.BlockSpec((BM, H), lambda i, be: (i, 0)),
    )
    return pl.pallas_call(
        _mlp_body,
        grid_spec=grid_spec,
        out_shape=jax.ShapeDtypeStruct((NB * BM, H), jnp.float32),
    )(block_expert, xs, Wg.astype(jnp.bfloat16), Wu.astype(jnp.bfloat16),
      Wd.astype(jnp.bfloat16))


_NC, _NS, _L = 2, 16, 16
_NW = _NC * _NS
_T = 4096
_PAIRS = _T * K
_CHUNK = _PAIRS // _NW
_ROWS = 32
_P = NB * BM
_NBP = 48


_HW = H // 2


def _dispatch_sc(xw, e_pairs, rank_pairs, starts16):
    mesh = plsc.VectorSubcoreMesh(core_axis_name="c", subcore_axis_name="s")
    ncc = _CHUNK // _ROWS

    @functools.partial(
        pl.kernel, mesh=mesh,
        out_type=[
            jax.ShapeDtypeStruct((_P, _HW), jnp.float32),
            jax.ShapeDtypeStruct((_NW, 8, _ROWS), jnp.int32),
        ],
        scratch_types=[
            pltpu.VMEM((_L,), jnp.int32),
            pltpu.VMEM((_CHUNK,), jnp.int32),
            pltpu.VMEM((_CHUNK,), jnp.int32),
            pltpu.VMEM((8, _ROWS), jnp.int32),
            pltpu.VMEM((8, _ROWS), jnp.int32),
            pltpu.VMEM((_ROWS, _HW), jnp.float32),
            pltpu.VMEM((_ROWS, _HW), jnp.float32),
            pltpu.SemaphoreType.DMA,
            pltpu.SemaphoreType.DMA,
        ],
        compiler_params=pltpu.CompilerParams(needs_layout_passes=False),
    )
    def k(x_hbm, e_hbm, r_hbm, st_hbm, xs_hbm, dest_hbm,
          start_v, ech_v, rch_v, tok_v, dst_v, rows_a, rows_b, sem_g, sem_s):
        wid = lax.axis_index("s") * _NC + lax.axis_index("c")
        base = wid * _CHUNK
        pltpu.sync_copy(st_hbm, start_v)
        pltpu.sync_copy(e_hbm.at[pl.ds(base, _CHUNK)], ech_v)
        pltpu.sync_copy(r_hbm.at[pl.ds(base, _CHUNK)], rch_v)
        iota = lax.iota(jnp.int32, _L)
        for j in range(_CHUNK // _L):
            e_vec = ech_v[pl.ds(j * _L, _L)]
            r_vec = rch_v[pl.ds(j * _L, _L)]
            d_vec = plsc.load_gather(start_v, [e_vec]) + r_vec
            dst_v[j // 2, pl.ds((j % 2) * _L, _L)] = d_vec
            pair = base + j * _L + iota
            tok_v[j // 2, pl.ds((j % 2) * _L, _L)] = pair >> 1
        pltpu.sync_copy(dst_v, dest_hbm.at[wid])
        bufs = [rows_a, rows_b]
        gh = [None] * ncc
        sh = [None] * ncc
        gh[0] = pltpu.async_copy(x_hbm.at[tok_v.at[0]], bufs[0], sem_g)
        for cc in range(ncc):
            gh[cc].wait()
            if cc >= 1:
                sh[cc - 1].wait()
            sh[cc] = pltpu.async_copy(
                bufs[cc % 2], xs_hbm.at[dst_v.at[cc]], sem_s)
            if cc + 1 < ncc:
                gh[cc + 1] = pltpu.async_copy(
                    x_hbm.at[tok_v.at[cc + 1]], bufs[(cc + 1) % 2], sem_g)
        sh[ncc - 1].wait()

    return k(xw, e_pairs, rank_pairs, starts16)


def _combine_sc(ys, w_pairs, dest3d):
    tok_per_w = _T // _NW
    sub = _ROWS // K
    mesh = plsc.VectorSubcoreMesh(core_axis_name="c", subcore_axis_name="s")

    @functools.partial(
        pl.kernel, mesh=mesh,
        out_type=jax.ShapeDtypeStruct((_T, H), jnp.float32),
        scratch_types=[
            pltpu.VMEM((8, _ROWS), jnp.int32),
            pltpu.VMEM((_CHUNK,), jnp.float32),
            pltpu.VMEM((_ROWS, H), jnp.float32),
            pltpu.VMEM((_ROWS, H), jnp.float32),
            pltpu.VMEM((sub, H), jnp.float32),
            pltpu.SemaphoreType.DMA,
        ],
        compiler_params=pltpu.CompilerParams(needs_layout_passes=False),
    )
    def k(ys_hbm, w_hbm, d_hbm, out_hbm, idx_v, w_v, rows_a, rows_b, out_v,
          sem):
        wid = lax.axis_index("s") * _NC + lax.axis_index("c")
        pltpu.sync_copy(d_hbm.at[wid], idx_v)
        pltpu.sync_copy(w_hbm.at[pl.ds(wid * _CHUNK, _CHUNK)], w_v)
        iota = lax.iota(jnp.int32, _L)
        ncc = _CHUNK // _ROWS
        bufs = [rows_a, rows_b]
        gh = [None] * ncc
        gh[0] = pltpu.async_copy(ys_hbm.at[idx_v.at[0]], bufs[0], sem)
        for cc in range(ncc):
            gh[cc].wait()
            if cc + 1 < ncc:
                gh[cc + 1] = pltpu.async_copy(
                    ys_hbm.at[idx_v.at[cc + 1]], bufs[(cc + 1) % 2], sem)
            rows_v = bufs[cc % 2]
            wa = w_v[pl.ds(cc * _ROWS, _L)]
            wb = w_v[pl.ds(cc * _ROWS + _L, _L)]
            for i in range(sub):
                wv = wa if i < sub // 2 else wb
                lane = 2 * i if i < sub // 2 else 2 * i - _L
                w0 = jnp.sum(jnp.where(iota == lane, wv, 0.0))
                w1 = jnp.sum(jnp.where(iota == lane + 1, wv, 0.0))

                def body(j, _):
                    y0 = rows_v[2 * i, pl.ds(j * _L, _L)]
                    y1 = rows_v[2 * i + 1, pl.ds(j * _L, _L)]
                    out_v[i, pl.ds(j * _L, _L)] = w0 * y0 + w1 * y1
                    return 0

                lax.fori_loop(0, H // _L, body, 0)
            pltpu.sync_copy(
                out_v, out_hbm.at[pl.ds(wid * tok_per_w + cc * sub, sub)])

    return k(ys, w_pairs, dest3d)


def kernel(hidden_states, router_w, Wg, Wu, Wd):
    b, s, _ = hidden_states.shape
    t = b * s
    x2d = hidden_states.reshape(t, H)

    logits, w2, e2, rank2, starts16, be = _router(x2d, router_w)

    xw = jax.lax.bitcast_convert_type(
        x2d.astype(jnp.bfloat16).reshape(t, _HW, 2), jnp.float32)
    xs_w, dest3d = _dispatch_sc(
        xw, e2.reshape(_PAIRS), rank2.reshape(_PAIRS), starts16.reshape(2 * E))
    xs = jax.lax.bitcast_convert_type(xs_w, jnp.bfloat16).reshape(_P, H)

    ys = _grouped_mlp(be.reshape(NBP)[:NB], xs, Wg, Wu, Wd)

    out = _combine_sc(ys, w2.reshape(_PAIRS), dest3d)
    return out.reshape(b, s, H), logits.reshape(b, s, E)


# speedup vs baseline: 1.0292x; 1.0292x over previous
import functools

import jax
import jax.numpy as jnp
from jax import lax
from jax.experimental import pallas as pl
from jax.experimental.pallas import tpu as pltpu
from jax.experimental.pallas import tpu_sc as plsc

E = 8
K = 2
H = 1024
F = 512
TB = 512
BM = 256
NB = 40
NBP = 48


def _router_body(x_ref, rw_ref, logits_ref, w_ref, e_ref, r_ref, start_ref,
                 be_ref, carry):
    i = pl.program_id(0)

    @pl.when(i == 0)
    def _():
        carry[...] = jnp.zeros_like(carry)

    x = x_ref[...]
    logits = jnp.dot(x, rw_ref[...], preferred_element_type=jnp.float32)
    logits_ref[...] = logits
    p = jax.nn.softmax(logits, axis=-1)
    iota8 = jax.lax.broadcasted_iota(jnp.int32, (TB, E), 1)
    m0 = jnp.max(p, axis=-1, keepdims=True)
    e0 = jnp.min(jnp.where(p == m0, iota8, E), axis=-1, keepdims=True)
    pm = jnp.where(iota8 == e0, -1.0, p)
    m1 = jnp.max(pm, axis=-1, keepdims=True)
    e1 = jnp.min(jnp.where(pm == m1, iota8, E), axis=-1, keepdims=True)
    s = m0 + m1
    w_ref[...] = jnp.concatenate([m0 / s, m1 / s], axis=-1)
    e_ref[...] = jnp.concatenate([e0, e1], axis=-1)

    oh0 = (iota8 == e0).astype(jnp.float32)
    oh1 = (iota8 == e1).astype(jnp.float32)
    tril = (jax.lax.broadcasted_iota(jnp.int32, (TB, TB), 0)
            > jax.lax.broadcasted_iota(jnp.int32, (TB, TB), 1)).astype(jnp.float32)
    c = carry[...]
    r0 = jnp.dot(tril, oh0, preferred_element_type=jnp.float32) + c
    c = c + jnp.sum(oh0, axis=0, keepdims=True)
    r1 = jnp.dot(tril, oh1, preferred_element_type=jnp.float32) + c
    c = c + jnp.sum(oh1, axis=0, keepdims=True)
    carry[...] = c
    rank0 = jnp.sum(oh0 * r0, axis=-1, keepdims=True)
    rank1 = jnp.sum(oh1 * r1, axis=-1, keepdims=True)
    r_ref[...] = jnp.concatenate([rank0, rank1], axis=-1).astype(jnp.int32)

    @pl.when(i == pl.num_programs(0) - 1)
    def _():
        padded = jnp.floor((c + (BM - 1)) / BM) * BM
        upper = (jax.lax.broadcasted_iota(jnp.int32, (E, E), 0)
                 <= jax.lax.broadcasted_iota(jnp.int32, (E, E), 1)
                 ).astype(jnp.float32)
        incl = jnp.dot(padded, upper, preferred_element_type=jnp.float32)
        starts = incl - padded
        start_ref[...] = jnp.concatenate(
            [starts, jnp.zeros((1, E), jnp.float32)], axis=-1).astype(jnp.int32)
        blk = (jax.lax.broadcasted_iota(jnp.int32, (NBP, E), 0)
               * BM).astype(jnp.float32)
        be_ref[...] = jnp.sum((blk >= incl).astype(jnp.int32),
                              axis=-1, keepdims=True)


def _router(x2d, router_w):
    t = x2d.shape[0]
    nblk = t // TB
    return pl.pallas_call(
        _router_body,
        grid=(nblk,),
        in_specs=[
            pl.BlockSpec((TB, H), lambda i: (i, 0)),
            pl.BlockSpec((H, E), lambda i: (0, 0)),
        ],
        out_specs=[
            pl.BlockSpec((TB, E), lambda i: (i, 0)),
            pl.BlockSpec((TB, K), lambda i: (i, 0)),
            pl.BlockSpec((TB, K), lambda i: (i, 0)),
            pl.BlockSpec((TB, K), lambda i: (i, 0)),
            pl.BlockSpec((1, 2 * E), lambda i: (0, 0)),
            pl.BlockSpec((NBP, 1), lambda i: (0, 0)),
        ],
        out_shape=[
            jax.ShapeDtypeStruct((t, E), jnp.float32),
            jax.ShapeDtypeStruct((t, K), jnp.float32),
            jax.ShapeDtypeStruct((t, K), jnp.int32),
            jax.ShapeDtypeStruct((t, K), jnp.int32),
            jax.ShapeDtypeStruct((1, 2 * E), jnp.int32),
            jax.ShapeDtypeStruct((NBP, 1), jnp.int32),
        ],
        scratch_shapes=[pltpu.VMEM((1, E), jnp.float32)],
    )(x2d, router_w)


def _mlp_body(be_ref, xs_ref, wg_ref, wu_ref, wd_ref, ys_ref):
    @pl.when(be_ref[pl.program_id(0)] < E)
    def _():
        x = xs_ref[...]
        wg = wg_ref[0].astype(jnp.bfloat16)
        wu = wu_ref[0].astype(jnp.bfloat16)
        wd = wd_ref[0].astype(jnp.bfloat16)
        g = jnp.dot(x, wg, preferred_element_type=jnp.float32)
        u = jnp.dot(x, wu, preferred_element_type=jnp.float32)
        h = (jax.nn.silu(g) * u).astype(jnp.bfloat16)
        ys_ref[...] = jnp.dot(h, wd, preferred_element_type=jnp.float32)


def _grouped_mlp(block_expert, xs, Wg, Wu, Wd):
    grid_spec = pltpu.PrefetchScalarGridSpec(
        num_scalar_prefetch=1,
        grid=(NB,),
        in_specs=[
            pl.BlockSpec((BM, H), lambda i, be: (i, 0)),
            pl.BlockSpec((1, H, F), lambda i, be: (jnp.minimum(be[i], E - 1), 0, 0)),
            pl.BlockSpec((1, H, F), lambda i, be: (jnp.minimum(be[i], E - 1), 0, 0)),
            pl.BlockSpec((1, F, H), lambda i, be: (jnp.minimum(be[i], E - 1), 0, 0)),
        ],
        out_specs=pl.BlockSpec((BM, H), lambda i, be: (i, 0)),
    )
    return pl.pallas_call(
        _mlp_body,
        grid_spec=grid_spec,
        out_shape=jax.ShapeDtypeStruct((NB * BM, H), jnp.float32),
    )(block_expert, xs, Wg, Wu, Wd)


_NC, _NS, _L = 2, 16, 16
_NW = _NC * _NS
_T = 4096
_PAIRS = _T * K
_CHUNK = _PAIRS // _NW
_ROWS = 32
_P = NB * BM
_NBP = 48


_HW = H // 2


def _dispatch_sc(xw, e_pairs, rank_pairs, starts16):
    mesh = plsc.VectorSubcoreMesh(core_axis_name="c", subcore_axis_name="s")
    ncc = _CHUNK // _ROWS

    @functools.partial(
        pl.kernel, mesh=mesh,
        out_type=[
            jax.ShapeDtypeStruct((_P, _HW), jnp.float32),
            jax.ShapeDtypeStruct((_NW, 8, _ROWS), jnp.int32),
        ],
        scratch_types=[
            pltpu.VMEM((_L,), jnp.int32),
            pltpu.VMEM((_CHUNK,), jnp.int32),
            pltpu.VMEM((_CHUNK,), jnp.int32),
            pltpu.VMEM((8, _ROWS), jnp.int32),
            pltpu.VMEM((8, _ROWS), jnp.int32),
            pltpu.VMEM((_ROWS, _HW), jnp.float32),
            pltpu.VMEM((_ROWS, _HW), jnp.float32),
            pltpu.SemaphoreType.DMA,
            pltpu.SemaphoreType.DMA,
        ],
        compiler_params=pltpu.CompilerParams(needs_layout_passes=False),
    )
    def k(x_hbm, e_hbm, r_hbm, st_hbm, xs_hbm, dest_hbm,
          start_v, ech_v, rch_v, tok_v, dst_v, rows_a, rows_b, sem_g, sem_s):
        wid = lax.axis_index("s") * _NC + lax.axis_index("c")
        base = wid * _CHUNK
        pltpu.sync_copy(st_hbm, start_v)
        pltpu.sync_copy(e_hbm.at[pl.ds(base, _CHUNK)], ech_v)
        pltpu.sync_copy(r_hbm.at[pl.ds(base, _CHUNK)], rch_v)
        iota = lax.iota(jnp.int32, _L)
        for j in range(_CHUNK // _L):
            e_vec = ech_v[pl.ds(j * _L, _L)]
            r_vec = rch_v[pl.ds(j * _L, _L)]
            d_vec = plsc.load_gather(start_v, [e_vec]) + r_vec
            dst_v[j // 2, pl.ds((j % 2) * _L, _L)] = d_vec
            pair = base + j * _L + iota
            tok_v[j // 2, pl.ds((j % 2) * _L, _L)] = pair >> 1
        pltpu.sync_copy(dst_v, dest_hbm.at[wid])
        bufs = [rows_a, rows_b]
        gh = [None] * ncc
        sh = [None] * ncc
        gh[0] = pltpu.async_copy(x_hbm.at[tok_v.at[0]], bufs[0], sem_g)
        for cc in range(ncc):
            gh[cc].wait()
            if cc >= 1:
                sh[cc - 1].wait()
            sh[cc] = pltpu.async_copy(
                bufs[cc % 2], xs_hbm.at[dst_v.at[cc]], sem_s)
            if cc + 1 < ncc:
                gh[cc + 1] = pltpu.async_copy(
                    x_hbm.at[tok_v.at[cc + 1]], bufs[(cc + 1) % 2], sem_g)
        sh[ncc - 1].wait()

    return k(xw, e_pairs, rank_pairs, starts16)


def _combine_sc(ys, w_pairs, dest3d):
    tok_per_w = _T // _NW
    sub = _ROWS // K
    mesh = plsc.VectorSubcoreMesh(core_axis_name="c", subcore_axis_name="s")

    @functools.partial(
        pl.kernel, mesh=mesh,
        out_type=jax.ShapeDtypeStruct((_T, H), jnp.float32),
        scratch_types=[
            pltpu.VMEM((8, _ROWS), jnp.int32),
            pltpu.VMEM((_CHUNK,), jnp.float32),
            pltpu.VMEM((_ROWS, H), jnp.float32),
            pltpu.VMEM((_ROWS, H), jnp.float32),
            pltpu.VMEM((sub, H), jnp.float32),
            pltpu.SemaphoreType.DMA,
        ],
        compiler_params=pltpu.CompilerParams(needs_layout_passes=False),
    )
    def k(ys_hbm, w_hbm, d_hbm, out_hbm, idx_v, w_v, rows_a, rows_b, out_v,
          sem):
        wid = lax.axis_index("s") * _NC + lax.axis_index("c")
        pltpu.sync_copy(d_hbm.at[wid], idx_v)
        pltpu.sync_copy(w_hbm.at[pl.ds(wid * _CHUNK, _CHUNK)], w_v)
        iota = lax.iota(jnp.int32, _L)
        ncc = _CHUNK // _ROWS
        bufs = [rows_a, rows_b]
        gh = [None] * ncc
        gh[0] = pltpu.async_copy(ys_hbm.at[idx_v.at[0]], bufs[0], sem)
        for cc in range(ncc):
            gh[cc].wait()
            if cc + 1 < ncc:
                gh[cc + 1] = pltpu.async_copy(
                    ys_hbm.at[idx_v.at[cc + 1]], bufs[(cc + 1) % 2], sem)
            rows_v = bufs[cc % 2]
            wa = w_v[pl.ds(cc * _ROWS, _L)]
            wb = w_v[pl.ds(cc * _ROWS + _L, _L)]
            for i in range(sub):
                wv = wa if i < sub // 2 else wb
                lane = 2 * i if i < sub // 2 else 2 * i - _L
                w0 = jnp.sum(jnp.where(iota == lane, wv, 0.0))
                w1 = jnp.sum(jnp.where(iota == lane + 1, wv, 0.0))

                def body(j, _):
                    y0 = rows_v[2 * i, pl.ds(j * _L, _L)]
                    y1 = rows_v[2 * i + 1, pl.ds(j * _L, _L)]
                    out_v[i, pl.ds(j * _L, _L)] = w0 * y0 + w1 * y1
                    return 0

                lax.fori_loop(0, H // _L, body, 0)
            pltpu.sync_copy(
                out_v, out_hbm.at[pl.ds(wid * tok_per_w + cc * sub, sub)])

    return k(ys, w_pairs, dest3d)


def kernel(hidden_states, router_w, Wg, Wu, Wd):
    b, s, _ = hidden_states.shape
    t = b * s
    x2d = hidden_states.reshape(t, H)

    logits, w2, e2, rank2, starts16, be = _router(x2d, router_w)

    xw = jax.lax.bitcast_convert_type(
        x2d.astype(jnp.bfloat16).reshape(t, _HW, 2), jnp.float32)
    xs_w, dest3d = _dispatch_sc(
        xw, e2.reshape(_PAIRS), rank2.reshape(_PAIRS), starts16.reshape(2 * E))
    xs = jax.lax.bitcast_convert_type(xs_w, jnp.bfloat16).reshape(_P, H)

    ys = _grouped_mlp(be.reshape(NBP)[:NB], xs, Wg, Wu, Wd)

    out = _combine_sc(ys, w2.reshape(_PAIRS), dest3d)
    return out.reshape(b, s, H), logits.reshape(b, s, E)


# speedup vs baseline: 2.4857x; 2.4153x over previous
import functools

import jax
import jax.numpy as jnp
from jax import lax
from jax.experimental import pallas as pl
from jax.experimental.pallas import tpu as pltpu
from jax.experimental.pallas import tpu_sc as plsc

E = 8
K = 2
H = 1024
F = 512
TB = 512
BM = 256
NB = 40
NBP = 48


def _router_body(x_ref, rw_ref, logits_ref, w_ref, e_ref, r_ref, start_ref,
                 be_ref, xw_ref, carry):
    i = pl.program_id(0)

    @pl.when(i == 0)
    def _():
        carry[...] = jnp.zeros_like(carry)

    x = x_ref[...]
    logits = jnp.dot(x, rw_ref[...], preferred_element_type=jnp.float32)
    logits_ref[...] = logits
    xi = pltpu.bitcast(x, jnp.int32)
    rt = (xi + 0x7FFF + ((xi >> 16) & 1)) >> 16
    xw_ref[...] = (rt[:, :H // 2] & 0xFFFF) | (rt[:, H // 2:] << 16)
    p = jax.nn.softmax(logits, axis=-1)
    iota8 = jax.lax.broadcasted_iota(jnp.int32, (TB, E), 1)
    m0 = jnp.max(p, axis=-1, keepdims=True)
    e0 = jnp.min(jnp.where(p == m0, iota8, E), axis=-1, keepdims=True)
    pm = jnp.where(iota8 == e0, -1.0, p)
    m1 = jnp.max(pm, axis=-1, keepdims=True)
    e1 = jnp.min(jnp.where(pm == m1, iota8, E), axis=-1, keepdims=True)
    s = m0 + m1
    w_ref[...] = jnp.concatenate([m0 / s, m1 / s], axis=-1)
    e_ref[...] = jnp.concatenate([e0, e1], axis=-1)

    oh0 = (iota8 == e0).astype(jnp.float32)
    oh1 = (iota8 == e1).astype(jnp.float32)
    tril = (jax.lax.broadcasted_iota(jnp.int32, (TB, TB), 0)
            > jax.lax.broadcasted_iota(jnp.int32, (TB, TB), 1)).astype(jnp.float32)
    c = carry[...]
    r0 = jnp.dot(tril, oh0, preferred_element_type=jnp.float32) + c
    c = c + jnp.sum(oh0, axis=0, keepdims=True)
    r1 = jnp.dot(tril, oh1, preferred_element_type=jnp.float32) + c
    c = c + jnp.sum(oh1, axis=0, keepdims=True)
    carry[...] = c
    rank0 = jnp.sum(oh0 * r0, axis=-1, keepdims=True)
    rank1 = jnp.sum(oh1 * r1, axis=-1, keepdims=True)
    r_ref[...] = jnp.concatenate([rank0, rank1], axis=-1).astype(jnp.int32)

    @pl.when(i == pl.num_programs(0) - 1)
    def _():
        padded = jnp.floor((c + (BM - 1)) / BM) * BM
        upper = (jax.lax.broadcasted_iota(jnp.int32, (E, E), 0)
                 <= jax.lax.broadcasted_iota(jnp.int32, (E, E), 1)
                 ).astype(jnp.float32)
        incl = jnp.dot(padded, upper, preferred_element_type=jnp.float32)
        starts = incl - padded
        start_ref[...] = jnp.concatenate(
            [starts, jnp.zeros((1, E), jnp.float32)], axis=-1).astype(jnp.int32)
        blk = (jax.lax.broadcasted_iota(jnp.int32, (NBP, E), 0)
               * BM).astype(jnp.float32)
        be_ref[...] = jnp.sum((blk >= incl).astype(jnp.int32),
                              axis=-1, keepdims=True)


def _router(x2d, router_w):
    t = x2d.shape[0]
    nblk = t // TB
    return pl.pallas_call(
        _router_body,
        grid=(nblk,),
        in_specs=[
            pl.BlockSpec((TB, H), lambda i: (i, 0)),
            pl.BlockSpec((H, E), lambda i: (0, 0)),
        ],
        out_specs=[
            pl.BlockSpec((TB, E), lambda i: (i, 0)),
            pl.BlockSpec((TB, K), lambda i: (i, 0)),
            pl.BlockSpec((TB, K), lambda i: (i, 0)),
            pl.BlockSpec((TB, K), lambda i: (i, 0)),
            pl.BlockSpec((1, 2 * E), lambda i: (0, 0)),
            pl.BlockSpec((NBP, 1), lambda i: (0, 0)),
            pl.BlockSpec((TB, H // 2), lambda i: (i, 0)),
        ],
        out_shape=[
            jax.ShapeDtypeStruct((t, E), jnp.float32),
            jax.ShapeDtypeStruct((t, K), jnp.float32),
            jax.ShapeDtypeStruct((t, K), jnp.int32),
            jax.ShapeDtypeStruct((t, K), jnp.int32),
            jax.ShapeDtypeStruct((1, 2 * E), jnp.int32),
            jax.ShapeDtypeStruct((NBP, 1), jnp.int32),
            jax.ShapeDtypeStruct((t, H // 2), jnp.int32),
        ],
        scratch_shapes=[pltpu.VMEM((1, E), jnp.float32)],
    )(x2d, router_w)


def _mlp_body(be_ref, xs_ref, wg_ref, wu_ref, wd_ref, ys_ref):
    @pl.when(be_ref[pl.program_id(0)] < E)
    def _():
        xwv = xs_ref[...]
        xlo = pltpu.bitcast(xwv << 16, jnp.float32).astype(jnp.bfloat16)
        xhi = pltpu.bitcast(xwv & jnp.int32(-65536),
                            jnp.float32).astype(jnp.bfloat16)
        wg = wg_ref[0].astype(jnp.bfloat16)
        wu = wu_ref[0].astype(jnp.bfloat16)
        wd = wd_ref[0].astype(jnp.bfloat16)
        hh = H // 2
        g = (jnp.dot(xlo, wg[:hh], preferred_element_type=jnp.float32)
             + jnp.dot(xhi, wg[hh:], preferred_element_type=jnp.float32))
        u = (jnp.dot(xlo, wu[:hh], preferred_element_type=jnp.float32)
             + jnp.dot(xhi, wu[hh:], preferred_element_type=jnp.float32))
        h = (jax.nn.silu(g) * u).astype(jnp.bfloat16)
        ys_ref[...] = jnp.dot(h, wd, preferred_element_type=jnp.float32)


def _grouped_mlp(block_expert, xs, Wg, Wu, Wd):
    grid_spec = pltpu.PrefetchScalarGridSpec(
        num_scalar_prefetch=1,
        grid=(NB,),
        in_specs=[
            pl.BlockSpec((BM, H // 2), lambda i, be: (i, 0)),
            pl.BlockSpec((1, H, F), lambda i, be: (jnp.minimum(be[i], E - 1), 0, 0)),
            pl.BlockSpec((1, H, F), lambda i, be: (jnp.minimum(be[i], E - 1), 0, 0)),
            pl.BlockSpec((1, F, H), lambda i, be: (jnp.minimum(be[i], E - 1), 0, 0)),
        ],
        out_specs=pl.BlockSpec((BM, H), lambda i, be: (i, 0)),
    )
    return pl.pallas_call(
        _mlp_body,
        grid_spec=grid_spec,
        out_shape=jax.ShapeDtypeStruct((NB * BM, H), jnp.float32),
    )(block_expert, xs, Wg, Wu, Wd)


_NC, _NS, _L = 2, 16, 16
_NW = _NC * _NS
_T = 4096
_PAIRS = _T * K
_CHUNK = _PAIRS // _NW
_ROWS = 32
_P = NB * BM
_NBP = 48


_HW = H // 2


def _dispatch_sc(xw, e_pairs, rank_pairs, starts16):
    mesh = plsc.VectorSubcoreMesh(core_axis_name="c", subcore_axis_name="s")
    ncc = _CHUNK // _ROWS

    @functools.partial(
        pl.kernel, mesh=mesh,
        out_type=[
            jax.ShapeDtypeStruct((_P, _HW), jnp.int32),
            jax.ShapeDtypeStruct((_NW, 8, _ROWS), jnp.int32),
        ],
        scratch_types=[
            pltpu.VMEM((_L,), jnp.int32),
            pltpu.VMEM((_CHUNK,), jnp.int32),
            pltpu.VMEM((_CHUNK,), jnp.int32),
            pltpu.VMEM((8, _ROWS), jnp.int32),
            pltpu.VMEM((8, _ROWS), jnp.int32),
            pltpu.VMEM((_ROWS, _HW), jnp.int32),
            pltpu.VMEM((_ROWS, _HW), jnp.int32),
            pltpu.SemaphoreType.DMA,
            pltpu.SemaphoreType.DMA,
        ],
        compiler_params=pltpu.CompilerParams(needs_layout_passes=False),
    )
    def k(x_hbm, e_hbm, r_hbm, st_hbm, xs_hbm, dest_hbm,
          start_v, ech_v, rch_v, tok_v, dst_v, rows_a, rows_b, sem_g, sem_s):
        wid = lax.axis_index("s") * _NC + lax.axis_index("c")
        base = wid * _CHUNK
        pltpu.sync_copy(st_hbm, start_v)
        pltpu.sync_copy(e_hbm.at[pl.ds(base, _CHUNK)], ech_v)
        pltpu.sync_copy(r_hbm.at[pl.ds(base, _CHUNK)], rch_v)
        iota = lax.iota(jnp.int32, _L)
        for j in range(_CHUNK // _L):
            e_vec = ech_v[pl.ds(j * _L, _L)]
            r_vec = rch_v[pl.ds(j * _L, _L)]
            d_vec = plsc.load_gather(start_v, [e_vec]) + r_vec
            dst_v[j // 2, pl.ds((j % 2) * _L, _L)] = d_vec
            pair = base + j * _L + iota
            tok_v[j // 2, pl.ds((j % 2) * _L, _L)] = pair >> 1
        pltpu.sync_copy(dst_v, dest_hbm.at[wid])
        bufs = [rows_a, rows_b]
        gh = [None] * ncc
        sh = [None] * ncc
        gh[0] = pltpu.async_copy(x_hbm.at[tok_v.at[0]], bufs[0], sem_g)
        for cc in range(ncc):
            gh[cc].wait()
            if cc >= 1:
                sh[cc - 1].wait()
            sh[cc] = pltpu.async_copy(
                bufs[cc % 2], xs_hbm.at[dst_v.at[cc]], sem_s)
            if cc + 1 < ncc:
                gh[cc + 1] = pltpu.async_copy(
                    x_hbm.at[tok_v.at[cc + 1]], bufs[(cc + 1) % 2], sem_g)
        sh[ncc - 1].wait()

    return k(xw, e_pairs, rank_pairs, starts16)


def _combine_sc(ys, w_pairs, dest3d):
    tok_per_w = _T // _NW
    sub = _ROWS // K
    mesh = plsc.VectorSubcoreMesh(core_axis_name="c", subcore_axis_name="s")

    @functools.partial(
        pl.kernel, mesh=mesh,
        out_type=jax.ShapeDtypeStruct((_T, H), jnp.float32),
        scratch_types=[
            pltpu.VMEM((8, _ROWS), jnp.int32),
            pltpu.VMEM((_CHUNK,), jnp.float32),
            pltpu.VMEM((_ROWS, H), jnp.float32),
            pltpu.VMEM((_ROWS, H), jnp.float32),
            pltpu.VMEM((sub, H), jnp.float32),
            pltpu.SemaphoreType.DMA,
        ],
        compiler_params=pltpu.CompilerParams(needs_layout_passes=False),
    )
    def k(ys_hbm, w_hbm, d_hbm, out_hbm, idx_v, w_v, rows_a, rows_b, out_v,
          sem):
        wid = lax.axis_index("s") * _NC + lax.axis_index("c")
        pltpu.sync_copy(d_hbm.at[wid], idx_v)
        pltpu.sync_copy(w_hbm.at[pl.ds(wid * _CHUNK, _CHUNK)], w_v)
        iota = lax.iota(jnp.int32, _L)
        ncc = _CHUNK // _ROWS
        bufs = [rows_a, rows_b]
        gh = [None] * ncc
        gh[0] = pltpu.async_copy(ys_hbm.at[idx_v.at[0]], bufs[0], sem)
        for cc in range(ncc):
            gh[cc].wait()
            if cc + 1 < ncc:
                gh[cc + 1] = pltpu.async_copy(
                    ys_hbm.at[idx_v.at[cc + 1]], bufs[(cc + 1) % 2], sem)
            rows_v = bufs[cc % 2]

            def tok_body(i, _):
                p0 = cc * _ROWS + 2 * i
                wv = w_v[pl.ds((p0 // _L) * _L, _L)]
                lane = p0 % _L
                w0 = jnp.sum(jnp.where(iota == lane, wv, 0.0))
                w1 = jnp.sum(jnp.where(iota == lane + 1, wv, 0.0))
                for j in range(H // _L):
                    y0 = rows_v[2 * i, pl.ds(j * _L, _L)]
                    y1 = rows_v[2 * i + 1, pl.ds(j * _L, _L)]
                    out_v[i, pl.ds(j * _L, _L)] = w0 * y0 + w1 * y1
                return 0

            lax.fori_loop(0, sub, tok_body, 0)
            pltpu.sync_copy(
                out_v, out_hbm.at[pl.ds(wid * tok_per_w + cc * sub, sub)])

    return k(ys, w_pairs, dest3d)


def kernel(hidden_states, router_w, Wg, Wu, Wd):
    b, s, _ = hidden_states.shape
    t = b * s
    x2d = hidden_states.reshape(t, H)

    logits, w2, e2, rank2, starts16, be, xw = _router(x2d, router_w)

    xs, dest3d = _dispatch_sc(
        xw, e2.reshape(_PAIRS), rank2.reshape(_PAIRS), starts16.reshape(2 * E))

    ys = _grouped_mlp(be.reshape(NBP)[:NB], xs, Wg, Wu, Wd)

    out = _combine_sc(ys, w2.reshape(_PAIRS), dest3d)
    return out.reshape(b, s, H), logits.reshape(b, s, E)


# speedup vs baseline: 3.0237x; 1.2164x over previous
import functools

import jax
import jax.numpy as jnp
from jax import lax
from jax.experimental import pallas as pl
from jax.experimental.pallas import tpu as pltpu
from jax.experimental.pallas import tpu_sc as plsc

E = 8
K = 2
H = 1024
F = 512
TB = 512
BM = 256
NB = 40
NBP = 48


def _router_body(x_ref, rw_ref, logits_ref, w_ref, e_ref, r_ref, start_ref,
                 be_ref, xw_ref, carry):
    i = pl.program_id(0)

    @pl.when(i == 0)
    def _():
        carry[...] = jnp.zeros_like(carry)

    x = x_ref[...]
    logits = jnp.dot(x, rw_ref[...], preferred_element_type=jnp.float32)
    logits_ref[...] = logits
    xi = pltpu.bitcast(x, jnp.int32)
    rt = (xi + 0x7FFF + ((xi >> 16) & 1)) >> 16
    xw_ref[...] = (rt[:, :H // 2] & 0xFFFF) | (rt[:, H // 2:] << 16)
    p = jax.nn.softmax(logits, axis=-1)
    iota8 = jax.lax.broadcasted_iota(jnp.int32, (TB, E), 1)
    m0 = jnp.max(p, axis=-1, keepdims=True)
    e0 = jnp.min(jnp.where(p == m0, iota8, E), axis=-1, keepdims=True)
    pm = jnp.where(iota8 == e0, -1.0, p)
    m1 = jnp.max(pm, axis=-1, keepdims=True)
    e1 = jnp.min(jnp.where(pm == m1, iota8, E), axis=-1, keepdims=True)
    s = m0 + m1
    w_ref[...] = jnp.concatenate([m0 / s, m1 / s], axis=-1)
    e_ref[...] = jnp.concatenate([e0, e1], axis=-1)

    oh0 = (iota8 == e0).astype(jnp.float32)
    oh1 = (iota8 == e1).astype(jnp.float32)
    tril = (jax.lax.broadcasted_iota(jnp.int32, (TB, TB), 0)
            > jax.lax.broadcasted_iota(jnp.int32, (TB, TB), 1)).astype(jnp.float32)
    c = carry[...]
    r0 = jnp.dot(tril, oh0, preferred_element_type=jnp.float32) + c
    c = c + jnp.sum(oh0, axis=0, keepdims=True)
    r1 = jnp.dot(tril, oh1, preferred_element_type=jnp.float32) + c
    c = c + jnp.sum(oh1, axis=0, keepdims=True)
    carry[...] = c
    rank0 = jnp.sum(oh0 * r0, axis=-1, keepdims=True)
    rank1 = jnp.sum(oh1 * r1, axis=-1, keepdims=True)
    r_ref[...] = jnp.concatenate([rank0, rank1], axis=-1).astype(jnp.int32)

    @pl.when(i == pl.num_programs(0) - 1)
    def _():
        padded = jnp.floor((c + (BM - 1)) / BM) * BM
        upper = (jax.lax.broadcasted_iota(jnp.int32, (E, E), 0)
                 <= jax.lax.broadcasted_iota(jnp.int32, (E, E), 1)
                 ).astype(jnp.float32)
        incl = jnp.dot(padded, upper, preferred_element_type=jnp.float32)
        starts = incl - padded
        start_ref[...] = jnp.concatenate(
            [starts, jnp.zeros((1, E), jnp.float32)], axis=-1).astype(jnp.int32)
        blk = (jax.lax.broadcasted_iota(jnp.int32, (NBP, E), 0)
               * BM).astype(jnp.float32)
        be_ref[...] = jnp.sum((blk >= incl).astype(jnp.int32),
                              axis=-1, keepdims=True)


def _router(x2d, router_w):
    t = x2d.shape[0]
    nblk = t // TB
    return pl.pallas_call(
        _router_body,
        grid=(nblk,),
        in_specs=[
            pl.BlockSpec((TB, H), lambda i: (i, 0)),
            pl.BlockSpec((H, E), lambda i: (0, 0)),
        ],
        out_specs=[
            pl.BlockSpec((TB, E), lambda i: (i, 0)),
            pl.BlockSpec((TB, K), lambda i: (i, 0)),
            pl.BlockSpec((TB, K), lambda i: (i, 0)),
            pl.BlockSpec((TB, K), lambda i: (i, 0)),
            pl.BlockSpec((1, 2 * E), lambda i: (0, 0)),
            pl.BlockSpec((NBP, 1), lambda i: (0, 0)),
            pl.BlockSpec((TB, H // 2), lambda i: (i, 0)),
        ],
        out_shape=[
            jax.ShapeDtypeStruct((t, E), jnp.float32),
            jax.ShapeDtypeStruct((t, K), jnp.float32),
            jax.ShapeDtypeStruct((t, K), jnp.int32),
            jax.ShapeDtypeStruct((t, K), jnp.int32),
            jax.ShapeDtypeStruct((1, 2 * E), jnp.int32),
            jax.ShapeDtypeStruct((NBP, 1), jnp.int32),
            jax.ShapeDtypeStruct((t, H // 2), jnp.int32),
        ],
        scratch_shapes=[pltpu.VMEM((1, E), jnp.float32)],
    )(x2d, router_w)


def _mlp_body(be_ref, xs_ref, wg_ref, wu_ref, wd_ref, ys_ref):
    @pl.when(be_ref[pl.program_id(0)] < E)
    def _():
        xwv = xs_ref[...]
        xlo = pltpu.bitcast(xwv << 16, jnp.float32).astype(jnp.bfloat16)
        xhi = pltpu.bitcast(xwv & jnp.int32(-65536),
                            jnp.float32).astype(jnp.bfloat16)
        wg = wg_ref[0].astype(jnp.bfloat16)
        wu = wu_ref[0].astype(jnp.bfloat16)
        wd = wd_ref[0].astype(jnp.bfloat16)
        hh = H // 2
        g = (jnp.dot(xlo, wg[:hh], preferred_element_type=jnp.float32)
             + jnp.dot(xhi, wg[hh:], preferred_element_type=jnp.float32))
        u = (jnp.dot(xlo, wu[:hh], preferred_element_type=jnp.float32)
             + jnp.dot(xhi, wu[hh:], preferred_element_type=jnp.float32))
        h = (jax.nn.silu(g) * u).astype(jnp.bfloat16)
        ys_ref[...] = jnp.dot(h, wd, preferred_element_type=jnp.float32)


def _grouped_mlp(block_expert, xs, Wg, Wu, Wd):
    grid_spec = pltpu.PrefetchScalarGridSpec(
        num_scalar_prefetch=1,
        grid=(NB,),
        in_specs=[
            pl.BlockSpec((BM, H // 2), lambda i, be: (i, 0)),
            pl.BlockSpec((1, H, F), lambda i, be: (jnp.minimum(be[i], E - 1), 0, 0)),
            pl.BlockSpec((1, H, F), lambda i, be: (jnp.minimum(be[i], E - 1), 0, 0)),
            pl.BlockSpec((1, F, H), lambda i, be: (jnp.minimum(be[i], E - 1), 0, 0)),
        ],
        out_specs=pl.BlockSpec((BM, H), lambda i, be: (i, 0)),
    )
    return pl.pallas_call(
        _mlp_body,
        grid_spec=grid_spec,
        out_shape=jax.ShapeDtypeStruct((NB * BM, H), jnp.float32),
    )(block_expert, xs, Wg, Wu, Wd)


_NC, _NS, _L = 2, 16, 16
_NW = _NC * _NS
_T = 4096
_PAIRS = _T * K
_CHUNK = _PAIRS // _NW
_ROWS = 32
_P = NB * BM
_NBP = 48


_HW = H // 2


def _dispatch_sc(xw, e_pairs, rank_pairs, starts16):
    mesh = plsc.VectorSubcoreMesh(core_axis_name="c", subcore_axis_name="s")
    ncc = _CHUNK // _ROWS

    @functools.partial(
        pl.kernel, mesh=mesh,
        out_type=[
            jax.ShapeDtypeStruct((_P, _HW), jnp.int32),
            jax.ShapeDtypeStruct((_NW, 8, _ROWS), jnp.int32),
        ],
        scratch_types=[
            pltpu.VMEM((_L,), jnp.int32),
            pltpu.VMEM((_CHUNK,), jnp.int32),
            pltpu.VMEM((_CHUNK,), jnp.int32),
            pltpu.VMEM((8, _ROWS), jnp.int32),
            pltpu.VMEM((8, _ROWS), jnp.int32),
            pltpu.VMEM((_ROWS, _HW), jnp.int32),
            pltpu.VMEM((_ROWS, _HW), jnp.int32),
            pltpu.SemaphoreType.DMA,
            pltpu.SemaphoreType.DMA,
        ],
        compiler_params=pltpu.CompilerParams(needs_layout_passes=False),
    )
    def k(x_hbm, e_hbm, r_hbm, st_hbm, xs_hbm, dest_hbm,
          start_v, ech_v, rch_v, tok_v, dst_v, rows_a, rows_b, sem_g, sem_s):
        wid = lax.axis_index("s") * _NC + lax.axis_index("c")
        base = wid * _CHUNK
        pltpu.sync_copy(st_hbm, start_v)
        pltpu.sync_copy(e_hbm.at[pl.ds(base, _CHUNK)], ech_v)
        pltpu.sync_copy(r_hbm.at[pl.ds(base, _CHUNK)], rch_v)
        iota = lax.iota(jnp.int32, _L)
        for j in range(_CHUNK // _L):
            e_vec = ech_v[pl.ds(j * _L, _L)]
            r_vec = rch_v[pl.ds(j * _L, _L)]
            d_vec = plsc.load_gather(start_v, [e_vec]) + r_vec
            dst_v[j // 2, pl.ds((j % 2) * _L, _L)] = d_vec
            pair = base + j * _L + iota
            tok_v[j // 2, pl.ds((j % 2) * _L, _L)] = pair >> 1
        pltpu.sync_copy(dst_v, dest_hbm.at[wid])
        bufs = [rows_a, rows_b]
        gh = [None] * ncc
        sh = [None] * ncc
        gh[0] = pltpu.async_copy(x_hbm.at[tok_v.at[0]], bufs[0], sem_g)
        for cc in range(ncc):
            gh[cc].wait()
            if cc >= 1:
                sh[cc - 1].wait()
            sh[cc] = pltpu.async_copy(
                bufs[cc % 2], xs_hbm.at[dst_v.at[cc]], sem_s)
            if cc + 1 < ncc:
                gh[cc + 1] = pltpu.async_copy(
                    x_hbm.at[tok_v.at[cc + 1]], bufs[(cc + 1) % 2], sem_g)
        sh[ncc - 1].wait()

    return k(xw, e_pairs, rank_pairs, starts16)


def _combine_sc(ys, w_pairs, dest3d):
    tok_per_w = _T // _NW
    sub = _ROWS // K
    mesh = plsc.VectorSubcoreMesh(core_axis_name="c", subcore_axis_name="s")

    @functools.partial(
        pl.kernel, mesh=mesh,
        out_type=jax.ShapeDtypeStruct((_T, H), jnp.float32),
        scratch_types=[
            pltpu.VMEM((8, _ROWS), jnp.int32),
            pltpu.VMEM((_CHUNK,), jnp.float32),
            pltpu.VMEM((_ROWS, H), jnp.float32),
            pltpu.VMEM((_ROWS, H), jnp.float32),
            pltpu.VMEM((sub, H), jnp.float32),
            pltpu.SemaphoreType.DMA,
        ],
        compiler_params=pltpu.CompilerParams(needs_layout_passes=False),
    )
    def k(ys_hbm, w_hbm, d_hbm, out_hbm, idx_v, w_v, rows_a, rows_b, out_v,
          sem):
        wid = lax.axis_index("s") * _NC + lax.axis_index("c")
        pltpu.sync_copy(d_hbm.at[wid], idx_v)
        pltpu.sync_copy(w_hbm.at[pl.ds(wid * _CHUNK, _CHUNK)], w_v)
        iota = lax.iota(jnp.int32, _L)
        ncc = _CHUNK // _ROWS
        bufs = [rows_a, rows_b]
        gh = [None] * ncc
        gh[0] = pltpu.async_copy(ys_hbm.at[idx_v.at[0]], bufs[0], sem)
        for cc in range(ncc):
            gh[cc].wait()
            if cc + 1 < ncc:
                gh[cc + 1] = pltpu.async_copy(
                    ys_hbm.at[idx_v.at[cc + 1]], bufs[(cc + 1) % 2], sem)
            rows_v = bufs[cc % 2]
            wa = w_v[pl.ds(cc * _ROWS, _L)]
            wb = w_v[pl.ds(cc * _ROWS + _L, _L)]
            for i in range(sub):
                wv = wa if i < sub // 2 else wb
                lane = 2 * i if i < sub // 2 else 2 * i - _L
                w0 = jnp.sum(jnp.where(iota == lane, wv, 0.0))
                w1 = jnp.sum(jnp.where(iota == lane + 1, wv, 0.0))

                def body(j):
                    y0 = rows_v[2 * i, pl.ds(j * _L, _L)]
                    y1 = rows_v[2 * i + 1, pl.ds(j * _L, _L)]
                    out_v[i, pl.ds(j * _L, _L)] = w0 * y0 + w1 * y1

                plsc.parallel_loop(0, H // _L, unroll=4)(body)
            pltpu.sync_copy(
                out_v, out_hbm.at[pl.ds(wid * tok_per_w + cc * sub, sub)])

    return k(ys, w_pairs, dest3d)


def kernel(hidden_states, router_w, Wg, Wu, Wd):
    b, s, _ = hidden_states.shape
    t = b * s
    x2d = hidden_states.reshape(t, H)

    logits, w2, e2, rank2, starts16, be, xw = _router(x2d, router_w)

    xs, dest3d = _dispatch_sc(
        xw, e2.reshape(_PAIRS), rank2.reshape(_PAIRS), starts16.reshape(2 * E))

    ys = _grouped_mlp(be.reshape(NBP)[:NB], xs, Wg, Wu, Wd)

    out = _combine_sc(ys, w2.reshape(_PAIRS), dest3d)
    return out.reshape(b, s, H), logits.reshape(b, s, E)


# speedup vs baseline: 3.0622x; 1.0128x over previous
import functools

import jax
import jax.numpy as jnp
from jax import lax
from jax.experimental import pallas as pl
from jax.experimental.pallas import tpu as pltpu
from jax.experimental.pallas import tpu_sc as plsc

E = 8
K = 2
H = 1024
F = 512
TB = 512
BM = 256
NB = 40
NBP = 48


def _router_body(x_ref, rw_ref, logits_ref, w_ref, e_ref, r_ref, start_ref,
                 be_ref, xw_ref, carry):
    i = pl.program_id(0)

    @pl.when(i == 0)
    def _():
        carry[...] = jnp.zeros_like(carry)

    x = x_ref[...]
    logits = jnp.dot(x, rw_ref[...], preferred_element_type=jnp.float32)
    logits_ref[...] = logits
    xi = pltpu.bitcast(x, jnp.int32)
    rt = (xi + 0x7FFF + ((xi >> 16) & 1)) >> 16
    xw_ref[...] = (rt[:, :H // 2] & 0xFFFF) | (rt[:, H // 2:] << 16)
    p = jax.nn.softmax(logits, axis=-1)
    iota8 = jax.lax.broadcasted_iota(jnp.int32, (TB, E), 1)
    m0 = jnp.max(p, axis=-1, keepdims=True)
    e0 = jnp.min(jnp.where(p == m0, iota8, E), axis=-1, keepdims=True)
    pm = jnp.where(iota8 == e0, -1.0, p)
    m1 = jnp.max(pm, axis=-1, keepdims=True)
    e1 = jnp.min(jnp.where(pm == m1, iota8, E), axis=-1, keepdims=True)
    s = m0 + m1
    w_ref[...] = jnp.concatenate([m0 / s, m1 / s], axis=-1)
    e_ref[...] = jnp.concatenate([e0, e1], axis=-1)

    oh0 = (iota8 == e0).astype(jnp.float32)
    oh1 = (iota8 == e1).astype(jnp.float32)
    tril = (jax.lax.broadcasted_iota(jnp.int32, (TB, TB), 0)
            > jax.lax.broadcasted_iota(jnp.int32, (TB, TB), 1)).astype(jnp.float32)
    c = carry[...]
    r0 = jnp.dot(tril, oh0, preferred_element_type=jnp.float32) + c
    c = c + jnp.sum(oh0, axis=0, keepdims=True)
    r1 = jnp.dot(tril, oh1, preferred_element_type=jnp.float32) + c
    c = c + jnp.sum(oh1, axis=0, keepdims=True)
    carry[...] = c
    rank0 = jnp.sum(oh0 * r0, axis=-1, keepdims=True)
    rank1 = jnp.sum(oh1 * r1, axis=-1, keepdims=True)
    r_ref[...] = jnp.concatenate([rank0, rank1], axis=-1).astype(jnp.int32)

    @pl.when(i == pl.num_programs(0) - 1)
    def _():
        padded = jnp.floor((c + (BM - 1)) / BM) * BM
        upper = (jax.lax.broadcasted_iota(jnp.int32, (E, E), 0)
                 <= jax.lax.broadcasted_iota(jnp.int32, (E, E), 1)
                 ).astype(jnp.float32)
        incl = jnp.dot(padded, upper, preferred_element_type=jnp.float32)
        starts = incl - padded
        start_ref[...] = jnp.concatenate(
            [starts, jnp.zeros((1, E), jnp.float32)], axis=-1).astype(jnp.int32)
        blk = (jax.lax.broadcasted_iota(jnp.int32, (NBP, E), 0)
               * BM).astype(jnp.float32)
        be_ref[...] = jnp.sum((blk >= incl).astype(jnp.int32),
                              axis=-1, keepdims=True)


def _router(x2d, router_w):
    t = x2d.shape[0]
    nblk = t // TB
    return pl.pallas_call(
        _router_body,
        grid=(nblk,),
        in_specs=[
            pl.BlockSpec((TB, H), lambda i: (i, 0)),
            pl.BlockSpec((H, E), lambda i: (0, 0)),
        ],
        out_specs=[
            pl.BlockSpec((TB, E), lambda i: (i, 0)),
            pl.BlockSpec((TB, K), lambda i: (i, 0)),
            pl.BlockSpec((TB, K), lambda i: (i, 0)),
            pl.BlockSpec((TB, K), lambda i: (i, 0)),
            pl.BlockSpec((1, 2 * E), lambda i: (0, 0)),
            pl.BlockSpec((NBP, 1), lambda i: (0, 0)),
            pl.BlockSpec((TB, H // 2), lambda i: (i, 0)),
        ],
        out_shape=[
            jax.ShapeDtypeStruct((t, E), jnp.float32),
            jax.ShapeDtypeStruct((t, K), jnp.float32),
            jax.ShapeDtypeStruct((t, K), jnp.int32),
            jax.ShapeDtypeStruct((t, K), jnp.int32),
            jax.ShapeDtypeStruct((1, 2 * E), jnp.int32),
            jax.ShapeDtypeStruct((NBP, 1), jnp.int32),
            jax.ShapeDtypeStruct((t, H // 2), jnp.int32),
        ],
        scratch_shapes=[pltpu.VMEM((1, E), jnp.float32)],
    )(x2d, router_w)


def _mlp_body(be_ref, xs_ref, wg_ref, wu_ref, wd_ref, ys_ref):
    @pl.when(be_ref[pl.program_id(0)] < E)
    def _():
        xwv = xs_ref[...]
        xlo = pltpu.bitcast(xwv << 16, jnp.float32).astype(jnp.bfloat16)
        xhi = pltpu.bitcast(xwv & jnp.int32(-65536),
                            jnp.float32).astype(jnp.bfloat16)
        wg = wg_ref[0].astype(jnp.bfloat16)
        wu = wu_ref[0].astype(jnp.bfloat16)
        wd = wd_ref[0].astype(jnp.bfloat16)
        hh = H // 2
        g = (jnp.dot(xlo, wg[:hh], preferred_element_type=jnp.float32)
             + jnp.dot(xhi, wg[hh:], preferred_element_type=jnp.float32))
        u = (jnp.dot(xlo, wu[:hh], preferred_element_type=jnp.float32)
             + jnp.dot(xhi, wu[hh:], preferred_element_type=jnp.float32))
        h = (jax.nn.silu(g) * u).astype(jnp.bfloat16)
        y = jnp.dot(h, wd, preferred_element_type=jnp.float32)
        yi = pltpu.bitcast(y, jnp.int32)
        rt = (yi + 0x7FFF + ((yi >> 16) & 1)) >> 16
        ys_ref[...] = (rt[:, :hh] & 0xFFFF) | (rt[:, hh:] << 16)


def _grouped_mlp(block_expert, xs, Wg, Wu, Wd):
    grid_spec = pltpu.PrefetchScalarGridSpec(
        num_scalar_prefetch=1,
        grid=(NB,),
        in_specs=[
            pl.BlockSpec((BM, H // 2), lambda i, be: (i, 0)),
            pl.BlockSpec((1, H, F), lambda i, be: (jnp.minimum(be[i], E - 1), 0, 0)),
            pl.BlockSpec((1, H, F), lambda i, be: (jnp.minimum(be[i], E - 1), 0, 0)),
            pl.BlockSpec((1, F, H), lambda i, be: (jnp.minimum(be[i], E - 1), 0, 0)),
        ],
        out_specs=pl.BlockSpec((BM, H // 2), lambda i, be: (i, 0)),
    )
    return pl.pallas_call(
        _mlp_body,
        grid_spec=grid_spec,
        out_shape=jax.ShapeDtypeStruct((NB * BM, H // 2), jnp.int32),
    )(block_expert, xs, Wg, Wu, Wd)


_NC, _NS, _L = 2, 16, 16
_NW = _NC * _NS
_T = 4096
_PAIRS = _T * K
_CHUNK = _PAIRS // _NW
_ROWS = 32
_P = NB * BM
_NBP = 48


_HW = H // 2


def _dispatch_sc(xw, e_pairs, rank_pairs, starts16):
    mesh = plsc.VectorSubcoreMesh(core_axis_name="c", subcore_axis_name="s")
    ncc = _CHUNK // _ROWS

    @functools.partial(
        pl.kernel, mesh=mesh,
        out_type=[
            jax.ShapeDtypeStruct((_P, _HW), jnp.int32),
            jax.ShapeDtypeStruct((_NW, 8, _ROWS), jnp.int32),
        ],
        scratch_types=[
            pltpu.VMEM((_L,), jnp.int32),
            pltpu.VMEM((_CHUNK,), jnp.int32),
            pltpu.VMEM((_CHUNK,), jnp.int32),
            pltpu.VMEM((8, _ROWS), jnp.int32),
            pltpu.VMEM((8, _ROWS), jnp.int32),
            pltpu.VMEM((_ROWS, _HW), jnp.int32),
            pltpu.VMEM((_ROWS, _HW), jnp.int32),
            pltpu.SemaphoreType.DMA,
            pltpu.SemaphoreType.DMA,
        ],
        compiler_params=pltpu.CompilerParams(needs_layout_passes=False),
    )
    def k(x_hbm, e_hbm, r_hbm, st_hbm, xs_hbm, dest_hbm,
          start_v, ech_v, rch_v, tok_v, dst_v, rows_a, rows_b, sem_g, sem_s):
        wid = lax.axis_index("s") * _NC + lax.axis_index("c")
        base = wid * _CHUNK
        pltpu.sync_copy(st_hbm, start_v)
        pltpu.sync_copy(e_hbm.at[pl.ds(base, _CHUNK)], ech_v)
        pltpu.sync_copy(r_hbm.at[pl.ds(base, _CHUNK)], rch_v)
        iota = lax.iota(jnp.int32, _L)
        for j in range(_CHUNK // _L):
            e_vec = ech_v[pl.ds(j * _L, _L)]
            r_vec = rch_v[pl.ds(j * _L, _L)]
            d_vec = plsc.load_gather(start_v, [e_vec]) + r_vec
            dst_v[j // 2, pl.ds((j % 2) * _L, _L)] = d_vec
            pair = base + j * _L + iota
            tok_v[j // 2, pl.ds((j % 2) * _L, _L)] = pair >> 1
        pltpu.sync_copy(dst_v, dest_hbm.at[wid])
        bufs = [rows_a, rows_b]
        gh = [None] * ncc
        sh = [None] * ncc
        gh[0] = pltpu.async_copy(x_hbm.at[tok_v.at[0]], bufs[0], sem_g)
        for cc in range(ncc):
            gh[cc].wait()
            if cc >= 1:
                sh[cc - 1].wait()
            sh[cc] = pltpu.async_copy(
                bufs[cc % 2], xs_hbm.at[dst_v.at[cc]], sem_s)
            if cc + 1 < ncc:
                gh[cc + 1] = pltpu.async_copy(
                    x_hbm.at[tok_v.at[cc + 1]], bufs[(cc + 1) % 2], sem_g)
        sh[ncc - 1].wait()

    return k(xw, e_pairs, rank_pairs, starts16)


def _combine_sc(ys, w_pairs, dest3d):
    tok_per_w = _T // _NW
    sub = _ROWS // K
    mesh = plsc.VectorSubcoreMesh(core_axis_name="c", subcore_axis_name="s")

    @functools.partial(
        pl.kernel, mesh=mesh,
        out_type=jax.ShapeDtypeStruct((_T, H), jnp.float32),
        scratch_types=[
            pltpu.VMEM((8, _ROWS), jnp.int32),
            pltpu.VMEM((_CHUNK,), jnp.float32),
            pltpu.VMEM((_ROWS, _HW), jnp.int32),
            pltpu.VMEM((_ROWS, _HW), jnp.int32),
            pltpu.VMEM((sub, H), jnp.float32),
            pltpu.SemaphoreType.DMA,
        ],
        compiler_params=pltpu.CompilerParams(needs_layout_passes=False),
    )
    def k(ys_hbm, w_hbm, d_hbm, out_hbm, idx_v, w_v, rows_a, rows_b, out_v,
          sem):
        wid = lax.axis_index("s") * _NC + lax.axis_index("c")
        pltpu.sync_copy(d_hbm.at[wid], idx_v)
        pltpu.sync_copy(w_hbm.at[pl.ds(wid * _CHUNK, _CHUNK)], w_v)
        iota = lax.iota(jnp.int32, _L)
        ncc = _CHUNK // _ROWS
        bufs = [rows_a, rows_b]
        gh = [None] * ncc
        gh[0] = pltpu.async_copy(ys_hbm.at[idx_v.at[0]], bufs[0], sem)
        for cc in range(ncc):
            gh[cc].wait()
            if cc + 1 < ncc:
                gh[cc + 1] = pltpu.async_copy(
                    ys_hbm.at[idx_v.at[cc + 1]], bufs[(cc + 1) % 2], sem)
            rows_v = bufs[cc % 2]
            wa = w_v[pl.ds(cc * _ROWS, _L)]
            wb = w_v[pl.ds(cc * _ROWS + _L, _L)]
            for i in range(sub):
                wv = wa if i < sub // 2 else wb
                lane = 2 * i if i < sub // 2 else 2 * i - _L
                w0 = jnp.sum(jnp.where(iota == lane, wv, 0.0))
                w1 = jnp.sum(jnp.where(iota == lane + 1, wv, 0.0))

                def body(j):
                    y0w = rows_v[2 * i, pl.ds(j * _L, _L)]
                    y1w = rows_v[2 * i + 1, pl.ds(j * _L, _L)]
                    m = jnp.int32(-65536)
                    y0lo = plsc.bitcast(y0w << 16, jnp.float32)
                    y1lo = plsc.bitcast(y1w << 16, jnp.float32)
                    y0hi = plsc.bitcast(y0w & m, jnp.float32)
                    y1hi = plsc.bitcast(y1w & m, jnp.float32)
                    out_v[i, pl.ds(j * _L, _L)] = w0 * y0lo + w1 * y1lo
                    out_v[i, pl.ds(_HW + j * _L, _L)] = w0 * y0hi + w1 * y1hi

                plsc.parallel_loop(0, _HW // _L, unroll=4)(body)
            pltpu.sync_copy(
                out_v, out_hbm.at[pl.ds(wid * tok_per_w + cc * sub, sub)])

    return k(ys, w_pairs, dest3d)


def kernel(hidden_states, router_w, Wg, Wu, Wd):
    b, s, _ = hidden_states.shape
    t = b * s
    x2d = hidden_states.reshape(t, H)

    logits, w2, e2, rank2, starts16, be, xw = _router(x2d, router_w)

    xs, dest3d = _dispatch_sc(
        xw, e2.reshape(_PAIRS), rank2.reshape(_PAIRS), starts16.reshape(2 * E))

    ys = _grouped_mlp(be.reshape(NBP)[:NB], xs, Wg, Wu, Wd)

    out = _combine_sc(ys, w2.reshape(_PAIRS), dest3d)
    return out.reshape(b, s, H), logits.reshape(b, s, E)
